# baseline (speedup 1.0000x reference)
import functools

import jax
import jax.numpy as jnp
from jax import lax
from jax.experimental import pallas as pl
from jax.experimental.pallas import tpu as pltpu
from jax.experimental.pallas import tpu_sc as plsc

_NC, _NS = 2, 16
_NW = _NC * _NS


def _vq_idx_body(z_ref, cb_ref, idx_ref):
    zb = z_ref[...].reshape(-1, z_ref.shape[-1])
    cb = cb_ref[...]
    K = cb.shape[0]
    BT = zb.shape[0]
    cross = lax.dot_general(
        zb, cb, (((1,), (1,)), ((), ())),
        preferred_element_type=jnp.float32)
    z_sq = jnp.sum(zb * zb, axis=-1, keepdims=True)
    e_sq = jnp.sum(cb * cb, axis=-1)
    C = 128
    TB = 64
    G = K // C
    liota = lax.broadcasted_iota(jnp.int32, (TB, C), 1)
    for t in range(BT // TB):
        zs = z_sq[t * TB:(t + 1) * TB, :]
        bd = zs - 2.0 * cross[t * TB:(t + 1) * TB, :C] + e_sq[None, :C]
        bg = jnp.zeros((TB, C), jnp.int32)
        for g in range(1, G):
            d = (zs - 2.0 * cross[t * TB:(t + 1) * TB, g * C:(g + 1) * C]
                 + e_sq[None, g * C:(g + 1) * C])
            lt = d < bd
            bd = jnp.where(lt, d, bd)
            bg = jnp.where(lt, g, bg)
        m = jnp.min(bd, axis=-1, keepdims=True)
        cand = jnp.where(bd == m, bg * C + liota, K)
        idx_ref[0, 0, pl.ds(t * TB, TB)] = jnp.min(cand, axis=-1)


def _sc_gather(codebook, idx3, N, D):
    b_per_w = N // _NW
    G3, _, BT3 = idx3.shape
    mesh = plsc.VectorSubcoreMesh(core_axis_name="c", subcore_axis_name="s")

    @functools.partial(
        pl.kernel,
        mesh=mesh,
        out_type=jax.ShapeDtypeStruct((N, D), jnp.float32),
        scratch_types=[
            pltpu.VMEM((b_per_w,), jnp.int32),
            pltpu.VMEM((b_per_w, D), jnp.float32),
            pltpu.SemaphoreType.DMA,
        ],
        compiler_params=pltpu.CompilerParams(use_tc_tiling_on_sc=False),
    )
    def gk(table_hbm, idx_hbm, out_hbm, idx_v, rows_v, sem):
        wid = lax.axis_index("s") * _NC + lax.axis_index("c")
        base = wid * b_per_w
        pltpu.sync_copy(
            idx_hbm.at[base // BT3, 0, pl.ds(base % BT3, b_per_w)], idx_v)
        pltpu.async_copy(table_hbm.at[idx_v], rows_v, sem).wait()
        pltpu.sync_copy(rows_v, out_hbm.at[pl.ds(base, b_per_w)])

    return gk(codebook, idx3)


def kernel(z_e, codebook):
    B, T, D = z_e.shape
    K = codebook.shape[0]
    N = B * T
    BB = 2
    BT = BB * T
    CH = 2
    BH = B // CH
    GH = BH // BB

    idx_parts, zq_parts = [], []
    for c in range(CH):
        idx3 = pl.pallas_call(
            _vq_idx_body,
            grid=(GH,),
            in_specs=[
                pl.BlockSpec((BB, T, D), lambda i, c=c: (c * GH + i, 0, 0)),
                pl.BlockSpec((K, D), lambda i: (0, 0)),
            ],
            out_specs=pl.BlockSpec((1, 1, BT), lambda i: (i, 0, 0)),
            out_shape=jax.ShapeDtypeStruct((GH, 1, BT), jnp.int32),
        )(z_e, codebook)
        idx_parts.append(idx3)
        zq_parts.append(_sc_gather(codebook, idx3, BH * T, D))

    idx = jnp.concatenate(idx_parts, axis=0).reshape(B, T)
    zq = jnp.concatenate(zq_parts, axis=0).reshape(B, T, D)
    return idx, zq
